# Initial kernel scaffold; baseline (speedup 1.0000x reference)
import jax
import jax.numpy as jnp
from jax.experimental import pallas as pl


def kernel(x, edge_attr, params, edge_index):
    raise NotImplementedError("write your pallas kernel here")


# speedup vs baseline: 1.9590x; 1.9590x over previous
import functools

import jax
import jax.numpy as jnp
from jax import lax
from jax.experimental import pallas as pl
from jax.experimental.pallas import tpu as pltpu
from jax.experimental.pallas import tpu_sc as plsc

_N = 10000
_E = 320000


def _mm_body(nadd, act, two_out, *refs):
    x = refs[0][...]
    for i in range(nadd):
        x = x + refs[1 + i][...]
    w = refs[1 + nadd][...]
    b = refs[2 + nadd][...]
    y = jnp.dot(x, w, preferred_element_type=jnp.float32) + b
    if act == "relu":
        y = jnp.maximum(y, 0.0)
    refs[3 + nadd][...] = y
    if two_out:
        refs[4 + nadd][...] = x


def _mm(x, w, b, act=None, extra_adds=(), block_rows=1000, out_sum=False):
    n, k = x.shape
    m = w.shape[1]
    nadd = len(extra_adds)
    grid = (n // block_rows,)
    row_spec = pl.BlockSpec((block_rows, k), lambda i: (i, 0))
    in_specs = [row_spec] * (1 + nadd) + [
        pl.BlockSpec((k, m), lambda i: (0, 0)),
        pl.BlockSpec((1, m), lambda i: (0, 0)),
    ]
    out_spec = pl.BlockSpec((block_rows, m), lambda i: (i, 0))
    out_shape = jax.ShapeDtypeStruct((n, m), jnp.float32)
    if out_sum:
        out_shape = (out_shape, jax.ShapeDtypeStruct((n, k), jnp.float32))
        out_specs = (out_spec, row_spec)
    else:
        out_specs = out_spec
    fn = pl.pallas_call(
        functools.partial(_mm_body, nadd, act, out_sum),
        grid=grid,
        in_specs=in_specs,
        out_specs=out_specs,
        out_shape=out_shape,
    )
    return fn(x, *extra_adds, w, b.reshape(1, m))


def _stats_body(has_resid, *refs):
    if has_resid:
        t = refs[0][...] + refs[1][...]
        refs[2][...] = t
        s_ref = refs[3]
    else:
        t = refs[0][...]
        s_ref = refs[1]

    @pl.when(pl.program_id(0) == 0)
    def _():
        s_ref[...] = jnp.zeros_like(s_ref)

    ps = jnp.sum(t, axis=0, keepdims=True)
    pq = jnp.sum(t * t, axis=0, keepdims=True)
    pad = jnp.zeros((6, t.shape[1]), jnp.float32)
    s_ref[...] += jnp.concatenate([ps, pq, pad], axis=0)


def _add_stats(a, r=None, block_rows=1000):
    n, c = a.shape
    grid = (n // block_rows,)
    row_spec = pl.BlockSpec((block_rows, c), lambda i: (i, 0))
    s_spec = pl.BlockSpec((8, c), lambda i: (0, 0))
    s_shape = jax.ShapeDtypeStruct((8, c), jnp.float32)
    if r is None:
        fn = pl.pallas_call(
            functools.partial(_stats_body, False),
            grid=grid,
            in_specs=[row_spec],
            out_specs=s_spec,
            out_shape=s_shape,
        )
        return a, fn(a)
    fn = pl.pallas_call(
        functools.partial(_stats_body, True),
        grid=grid,
        in_specs=[row_spec, row_spec],
        out_specs=(row_spec, s_spec),
        out_shape=(jax.ShapeDtypeStruct((n, c), jnp.float32), s_shape),
    )
    return fn(a, r)


def _bn_body(relu, n, *refs):
    x_ref, s_ref, g_ref, b_ref, o_ref = refs
    s = s_ref[...]
    mu = s[0:1, :] / n
    var = s[1:2, :] / n - mu * mu
    rstd = lax.rsqrt(var + 1e-5)
    y = (x_ref[...] - mu) * (rstd * g_ref[...]) + b_ref[...]
    if relu:
        y = jnp.maximum(y, 0.0)
    o_ref[...] = y


def _bn_apply(x, s, g, b, relu=False, block_rows=1000):
    n, c = x.shape
    grid = (n // block_rows,)
    row_spec = pl.BlockSpec((block_rows, c), lambda i: (i, 0))
    vec_spec = pl.BlockSpec((1, c), lambda i: (0, 0))
    fn = pl.pallas_call(
        lambda *refs: _bn_body(relu, float(n), *refs),
        grid=grid,
        in_specs=[row_spec, pl.BlockSpec((8, c), lambda i: (0, 0)),
                  vec_spec, vec_spec],
        out_specs=row_spec,
        out_shape=jax.ShapeDtypeStruct((n, c), jnp.float32),
    )
    return fn(x, s, g.reshape(1, c), b.reshape(1, c))


def _attn_body(scale, q_ref, k_ref, v_ref, o_ref):
    q = q_ref[0]
    k = k_ref[0]
    s = lax.dot_general(q, k, (((1,), (1,)), ((), ())),
                        preferred_element_type=jnp.float32) * scale
    m = jnp.max(s, axis=1, keepdims=True)
    p = jnp.exp(s - m)
    denom = jnp.sum(p, axis=1, keepdims=True)
    o = lax.dot_general(p, v_ref[0], (((1,), (0,)), ((), ())),
                        preferred_element_type=jnp.float32)
    o_ref[0] = o / denom


def _attention(q, k, v, heads, block_q=400):
    n, c = q.shape
    dh = c // heads
    scale = float(dh) ** -0.5
    qh = q.reshape(n, heads, dh).transpose(1, 0, 2)
    kh = k.reshape(n, heads, dh).transpose(1, 0, 2)
    vh = v.reshape(n, heads, dh).transpose(1, 0, 2)
    grid = (heads, n // block_q)
    q_spec = pl.BlockSpec((1, block_q, dh), lambda h, i: (h, i, 0))
    kv_spec = pl.BlockSpec((1, n, dh), lambda h, i: (h, 0, 0))
    fn = pl.pallas_call(
        functools.partial(_attn_body, scale),
        grid=grid,
        in_specs=[q_spec, kv_spec, kv_spec],
        out_specs=q_spec,
        out_shape=jax.ShapeDtypeStruct((heads, n, dh), jnp.float32),
    )
    oh = fn(qh, kh, vh)
    return oh.transpose(1, 0, 2).reshape(n, c)


_CH = 80
_ROWS_PER_SUB = 624


def _gine_body(x_hbm, ea_hbm, src_hbm, dst_hbm, z_hbm, out_hbm,
               sidx, didx, rows, eab, agg_sh, sem):
    cid = lax.axis_index("c")
    sid = lax.axis_index("s")
    wid = sid * 2 + cid
    e0 = wid * (_E // 32)

    r0 = sid * _ROWS_PER_SUB
    tail = 16 * _ROWS_PER_SUB
    pltpu.sync_copy(z_hbm.at[pl.ds(r0, _ROWS_PER_SUB)],
                    agg_sh.at[pl.ds(r0, _ROWS_PER_SUB)])

    @pl.when(sid == 15)
    def _():
        pltpu.sync_copy(z_hbm.at[pl.ds(tail, _N - tail)],
                        agg_sh.at[pl.ds(tail, _N - tail)])

    plsc.subcore_barrier()

    nchunks = (_E // 32) // _CH

    def chunk(ci, carry):
        base = e0 + ci * _CH
        pltpu.sync_copy(src_hbm.at[pl.ds(base, _CH)], sidx)
        pltpu.sync_copy(dst_hbm.at[pl.ds(base, _CH)], didx)
        pltpu.async_copy(x_hbm.at[sidx], rows, sem).wait()
        pltpu.sync_copy(ea_hbm.at[pl.ds(base, _CH)], eab)

        def edge(i, c2):
            for j in range(8):
                sl = pl.ds(j * 16, 16)
                rows[i, sl] = jnp.maximum(rows[i, sl] + eab[i, sl], 0.0)
            return c2

        lax.fori_loop(0, _CH, edge, 0)
        pltpu.sync_copy(rows, agg_sh.at[didx], add=True)
        return carry

    lax.fori_loop(0, nchunks, chunk, 0)
    plsc.subcore_barrier()

    pltpu.sync_copy(agg_sh.at[pl.ds(r0, _ROWS_PER_SUB)],
                    out_hbm.at[pl.ds(cid * _N + r0, _ROWS_PER_SUB)])

    @pl.when(sid == 15)
    def _():
        pltpu.sync_copy(agg_sh.at[pl.ds(tail, _N - tail)],
                        out_hbm.at[pl.ds(cid * _N + tail, _N - tail)])


def _gine_sc(x, ea, src, dst):
    c = x.shape[1]
    mesh = plsc.VectorSubcoreMesh(core_axis_name="c", subcore_axis_name="s")
    fn = functools.partial(
        pl.kernel,
        mesh=mesh,
        out_type=jax.ShapeDtypeStruct((2 * _N, c), jnp.float32),
        scratch_types=[
            pltpu.VMEM((_CH,), jnp.int32),
            pltpu.VMEM((_CH,), jnp.int32),
            pltpu.VMEM((_CH, c), jnp.float32),
            pltpu.VMEM((_CH, c), jnp.float32),
            pltpu.VMEM_SHARED((_N, c), jnp.float32),
            pltpu.SemaphoreType.DMA,
        ],
    )(_gine_body)
    zeros = jnp.zeros((_N, c), jnp.float32)
    return fn(x, ea, src, dst, zeros)


def _gps_layer(x, ea, src, dst, p, heads):
    aggs = _gine_sc(x, ea, src, dst)
    y1 = _mm(x, p["W1"], p["b1"], act="relu",
             extra_adds=(aggs[:_N], aggs[_N:]))
    h2 = _mm(y1, p["W2"], p["b2"])
    t1, s1 = _add_stats(h2, x)
    hn = _bn_apply(t1, s1, p["n1_g"], p["n1_b"])

    q = _mm(x, p["Wq"], p["bq"])
    k = _mm(x, p["Wk"], p["bk"])
    v = _mm(x, p["Wv"], p["bv"])
    ao = _attention(q, k, v, heads)
    ha = _mm(ao, p["Wo"], p["bo"])
    t2, s2 = _add_stats(ha, x)
    han = _bn_apply(t2, s2, p["n2_g"], p["n2_b"])

    y, out = _mm(hn, p["mW1"], p["mb1"], act="relu", extra_adds=(han,),
                 out_sum=True)
    r2 = _mm(y, p["mW2"], p["mb2"])
    t3, s3 = _add_stats(r2, out)
    return _bn_apply(t3, s3, p["n3_g"], p["n3_b"])


def _bn_relu(x, g, b):
    _, s = _add_stats(x)
    return _bn_apply(x, s, g, b, relu=True)


def kernel(x, edge_attr, params, edge_index):
    p1, p2 = params["gps1"], params["gps2"]
    src = edge_index[0]
    dst = edge_index[1]
    ea1 = _mm(edge_attr, p1["We"], p1["be"], block_rows=4000)
    ea2 = _mm(edge_attr, p2["We"], p2["be"], block_rows=4000)
    h = _gps_layer(x, ea1, src, dst, p1, heads=2)
    h = _bn_relu(h, params["bn1_g"], params["bn1_b"])
    h = _mm(h, params["lin1_W"], params["lin1_b"])
    h = _bn_relu(h, params["bn2_g"], params["bn2_b"])
    h = _gps_layer(h, ea2, src, dst, p2, heads=1)
    h = _bn_relu(h, params["bn2_g"], params["bn2_b"])
    return _mm(h, params["lin2_W"], params["lin2_b"])


# speedup vs baseline: 2.2134x; 1.1299x over previous
import functools

import jax
import jax.numpy as jnp
from jax import lax
from jax.experimental import pallas as pl
from jax.experimental.pallas import tpu as pltpu
from jax.experimental.pallas import tpu_sc as plsc

_N = 10000
_E = 320000


def _mm_body(nadd, act, two_out, *refs):
    x = refs[0][...]
    for i in range(nadd):
        x = x + refs[1 + i][...]
    w = refs[1 + nadd][...]
    b = refs[2 + nadd][...]
    y = jnp.dot(x, w, preferred_element_type=jnp.float32) + b
    if act == "relu":
        y = jnp.maximum(y, 0.0)
    refs[3 + nadd][...] = y
    if two_out:
        refs[4 + nadd][...] = x


def _mm(x, w, b, act=None, extra_adds=(), block_rows=1000, out_sum=False):
    n, k = x.shape
    m = w.shape[1]
    nadd = len(extra_adds)
    grid = (n // block_rows,)
    row_spec = pl.BlockSpec((block_rows, k), lambda i: (i, 0))
    in_specs = [row_spec] * (1 + nadd) + [
        pl.BlockSpec((k, m), lambda i: (0, 0)),
        pl.BlockSpec((1, m), lambda i: (0, 0)),
    ]
    out_spec = pl.BlockSpec((block_rows, m), lambda i: (i, 0))
    out_shape = jax.ShapeDtypeStruct((n, m), jnp.float32)
    if out_sum:
        out_shape = (out_shape, jax.ShapeDtypeStruct((n, k), jnp.float32))
        out_specs = (out_spec, row_spec)
    else:
        out_specs = out_spec
    fn = pl.pallas_call(
        functools.partial(_mm_body, nadd, act, out_sum),
        grid=grid,
        in_specs=in_specs,
        out_specs=out_specs,
        out_shape=out_shape,
    )
    return fn(x, *extra_adds, w, b.reshape(1, m))


def _mm3_body(x_ref, w_ref, b_ref, o1_ref, o2_ref, o3_ref):
    y = jnp.dot(x_ref[...], w_ref[...],
                preferred_element_type=jnp.float32) + b_ref[...]
    m = o1_ref.shape[1]
    o1_ref[...] = y[:, :m]
    o2_ref[...] = y[:, m:2 * m]
    o3_ref[...] = y[:, 2 * m:]


def _mm3(x, ws, bs, block_rows=1000):
    n, k = x.shape
    m = ws[0].shape[1]
    w3 = jnp.concatenate(ws, axis=1)
    b3 = jnp.concatenate(bs).reshape(1, 3 * m)
    grid = (n // block_rows,)
    out_spec = pl.BlockSpec((block_rows, m), lambda i: (i, 0))
    fn = pl.pallas_call(
        _mm3_body,
        grid=grid,
        in_specs=[pl.BlockSpec((block_rows, k), lambda i: (i, 0)),
                  pl.BlockSpec((k, 3 * m), lambda i: (0, 0)),
                  pl.BlockSpec((1, 3 * m), lambda i: (0, 0))],
        out_specs=(out_spec, out_spec, out_spec),
        out_shape=tuple(jax.ShapeDtypeStruct((n, m), jnp.float32)
                        for _ in range(3)),
    )
    return fn(x, w3, b3)


def _stats_body(has_resid, *refs):
    if has_resid:
        t = refs[0][...] + refs[1][...]
        refs[2][...] = t
        s_ref = refs[3]
    else:
        t = refs[0][...]
        s_ref = refs[1]

    @pl.when(pl.program_id(0) == 0)
    def _():
        s_ref[...] = jnp.zeros_like(s_ref)

    ps = jnp.sum(t, axis=0, keepdims=True)
    pq = jnp.sum(t * t, axis=0, keepdims=True)
    pad = jnp.zeros((6, t.shape[1]), jnp.float32)
    s_ref[...] += jnp.concatenate([ps, pq, pad], axis=0)


def _add_stats(a, r=None, block_rows=1000):
    n, c = a.shape
    grid = (n // block_rows,)
    row_spec = pl.BlockSpec((block_rows, c), lambda i: (i, 0))
    s_spec = pl.BlockSpec((8, c), lambda i: (0, 0))
    s_shape = jax.ShapeDtypeStruct((8, c), jnp.float32)
    if r is None:
        fn = pl.pallas_call(
            functools.partial(_stats_body, False),
            grid=grid,
            in_specs=[row_spec],
            out_specs=s_spec,
            out_shape=s_shape,
        )
        return a, fn(a)
    fn = pl.pallas_call(
        functools.partial(_stats_body, True),
        grid=grid,
        in_specs=[row_spec, row_spec],
        out_specs=(row_spec, s_spec),
        out_shape=(jax.ShapeDtypeStruct((n, c), jnp.float32), s_shape),
    )
    return fn(a, r)


def _bn_body(relu, n, *refs):
    x_ref, s_ref, g_ref, b_ref, o_ref = refs
    s = s_ref[...]
    mu = s[0:1, :] / n
    var = s[1:2, :] / n - mu * mu
    rstd = lax.rsqrt(var + 1e-5)
    y = (x_ref[...] - mu) * (rstd * g_ref[...]) + b_ref[...]
    if relu:
        y = jnp.maximum(y, 0.0)
    o_ref[...] = y


def _bn_apply(x, s, g, b, relu=False, block_rows=1000):
    n, c = x.shape
    grid = (n // block_rows,)
    row_spec = pl.BlockSpec((block_rows, c), lambda i: (i, 0))
    vec_spec = pl.BlockSpec((1, c), lambda i: (0, 0))
    fn = pl.pallas_call(
        lambda *refs: _bn_body(relu, float(n), *refs),
        grid=grid,
        in_specs=[row_spec, pl.BlockSpec((8, c), lambda i: (0, 0)),
                  vec_spec, vec_spec],
        out_specs=row_spec,
        out_shape=jax.ShapeDtypeStruct((n, c), jnp.float32),
    )
    return fn(x, s, g.reshape(1, c), b.reshape(1, c))


def _attn_body(scale, q_ref, k_ref, v_ref, o_ref):
    q = q_ref[0]
    k = k_ref[0]
    s = lax.dot_general(q, k, (((1,), (1,)), ((), ())),
                        preferred_element_type=jnp.float32) * scale
    m = jnp.max(s, axis=1, keepdims=True)
    p = jnp.exp(s - m)
    denom = jnp.sum(p, axis=1, keepdims=True)
    o = lax.dot_general(p, v_ref[0], (((1,), (0,)), ((), ())),
                        preferred_element_type=jnp.float32)
    o_ref[0] = o / denom


def _attention(q, k, v, heads, block_q=400):
    n, c = q.shape
    dh = c // heads
    scale = float(dh) ** -0.5
    qh = q.reshape(n, heads, dh).transpose(1, 0, 2)
    kh = k.reshape(n, heads, dh).transpose(1, 0, 2)
    vh = v.reshape(n, heads, dh).transpose(1, 0, 2)
    grid = (heads, n // block_q)
    q_spec = pl.BlockSpec((1, block_q, dh), lambda h, i: (h, i, 0))
    kv_spec = pl.BlockSpec((1, n, dh), lambda h, i: (h, 0, 0))
    fn = pl.pallas_call(
        functools.partial(_attn_body, scale),
        grid=grid,
        in_specs=[q_spec, kv_spec, kv_spec],
        out_specs=q_spec,
        out_shape=jax.ShapeDtypeStruct((heads, n, dh), jnp.float32),
    )
    oh = fn(qh, kh, vh)
    return oh.transpose(1, 0, 2).reshape(n, c)


_CH = 80
_ROWS_PER_SUB = 624


def _gine_body(x_hbm, ea_hbm, src_hbm, dst_hbm, z_hbm, out_hbm,
               sidx0, didx0, rows0, eab0, sidx1, didx1, rows1, eab1,
               agg_sh, lsem0, gsem0, lsem1, gsem1):
    cid = lax.axis_index("c")
    sid = lax.axis_index("s")
    wid = sid * 2 + cid
    e0 = wid * (_E // 32)
    bufA = (sidx0, didx0, rows0, eab0, lsem0, gsem0)
    bufB = (sidx1, didx1, rows1, eab1, lsem1, gsem1)

    def issue_loads(c, bs):
        sidx, didx, rows, eab, lsem, gsem = bs
        base = e0 + c * _CH
        pltpu.async_copy(src_hbm.at[pl.ds(base, _CH)], sidx, lsem)
        pltpu.async_copy(dst_hbm.at[pl.ds(base, _CH)], didx, lsem)
        pltpu.async_copy(ea_hbm.at[pl.ds(base, _CH)], eab, lsem)

    def wait_loads(c, bs):
        sidx, didx, rows, eab, lsem, gsem = bs
        base = e0 + c * _CH
        pltpu.make_async_copy(src_hbm.at[pl.ds(base, _CH)], sidx, lsem).wait()
        pltpu.make_async_copy(dst_hbm.at[pl.ds(base, _CH)], didx, lsem).wait()
        pltpu.make_async_copy(ea_hbm.at[pl.ds(base, _CH)], eab, lsem).wait()

    def issue_gather(bs):
        sidx, didx, rows, eab, lsem, gsem = bs
        pltpu.async_copy(x_hbm.at[sidx], rows, gsem)

    def wait_gather(bs):
        sidx, didx, rows, eab, lsem, gsem = bs
        pltpu.make_async_copy(x_hbm.at[sidx], rows, gsem).wait()

    def compute_scatter(bs):
        sidx, didx, rows, eab, lsem, gsem = bs

        def edge4(i, carry):
            for u in range(4):
                r = i * 4 + u
                for j in range(8):
                    sl = pl.ds(j * 16, 16)
                    rows[r, sl] = jnp.maximum(rows[r, sl] + eab[r, sl], 0.0)
            return carry

        lax.fori_loop(0, _CH // 4, edge4, 0)
        pltpu.sync_copy(rows, agg_sh.at[didx], add=True)

    r0 = sid * _ROWS_PER_SUB
    tail = 16 * _ROWS_PER_SUB
    pltpu.sync_copy(z_hbm.at[pl.ds(r0, _ROWS_PER_SUB)],
                    agg_sh.at[pl.ds(r0, _ROWS_PER_SUB)])

    @pl.when(sid == 15)
    def _():
        pltpu.sync_copy(z_hbm.at[pl.ds(tail, _N - tail)],
                        agg_sh.at[pl.ds(tail, _N - tail)])

    plsc.subcore_barrier()

    nchunks = (_E // 32) // _CH
    npairs = (nchunks - 1) // 2

    issue_loads(0, bufA)
    wait_loads(0, bufA)
    issue_gather(bufA)
    issue_loads(1, bufB)

    def pair(h, carry):
        c = 2 * h
        wait_loads(c + 1, bufB)
        issue_gather(bufB)
        wait_gather(bufA)
        compute_scatter(bufA)
        issue_loads(c + 2, bufA)
        wait_loads(c + 2, bufA)
        issue_gather(bufA)
        wait_gather(bufB)
        compute_scatter(bufB)

        @pl.when(h < npairs - 1)
        def _():
            issue_loads(c + 3, bufB)

        return carry

    lax.fori_loop(0, npairs, pair, 0)
    wait_gather(bufA)
    compute_scatter(bufA)
    plsc.subcore_barrier()

    pltpu.sync_copy(agg_sh.at[pl.ds(r0, _ROWS_PER_SUB)],
                    out_hbm.at[pl.ds(cid * _N + r0, _ROWS_PER_SUB)])

    @pl.when(sid == 15)
    def _():
        pltpu.sync_copy(agg_sh.at[pl.ds(tail, _N - tail)],
                        out_hbm.at[pl.ds(cid * _N + tail, _N - tail)])


def _gine_sc(x, ea, src, dst):
    c = x.shape[1]
    mesh = plsc.VectorSubcoreMesh(core_axis_name="c", subcore_axis_name="s")
    fn = functools.partial(
        pl.kernel,
        mesh=mesh,
        out_type=jax.ShapeDtypeStruct((2 * _N, c), jnp.float32),
        scratch_types=[
            pltpu.VMEM((_CH,), jnp.int32),
            pltpu.VMEM((_CH,), jnp.int32),
            pltpu.VMEM((_CH, c), jnp.float32),
            pltpu.VMEM((_CH, c), jnp.float32),
            pltpu.VMEM((_CH,), jnp.int32),
            pltpu.VMEM((_CH,), jnp.int32),
            pltpu.VMEM((_CH, c), jnp.float32),
            pltpu.VMEM((_CH, c), jnp.float32),
            pltpu.VMEM_SHARED((_N, c), jnp.float32),
            pltpu.SemaphoreType.DMA,
            pltpu.SemaphoreType.DMA,
            pltpu.SemaphoreType.DMA,
            pltpu.SemaphoreType.DMA,
        ],
    )(_gine_body)
    zeros = jnp.zeros((_N, c), jnp.float32)
    return fn(x, ea, src, dst, zeros)


def _gps_layer(x, ea, src, dst, p, heads):
    aggs = _gine_sc(x, ea, src, dst)
    y1 = _mm(x, p["W1"], p["b1"], act="relu",
             extra_adds=(aggs[:_N], aggs[_N:]))
    h2 = _mm(y1, p["W2"], p["b2"])
    t1, s1 = _add_stats(h2, x)
    hn = _bn_apply(t1, s1, p["n1_g"], p["n1_b"])

    q, k, v = _mm3(x, (p["Wq"], p["Wk"], p["Wv"]), (p["bq"], p["bk"], p["bv"]))
    ao = _attention(q, k, v, heads)
    ha = _mm(ao, p["Wo"], p["bo"])
    t2, s2 = _add_stats(ha, x)
    han = _bn_apply(t2, s2, p["n2_g"], p["n2_b"])

    y, out = _mm(hn, p["mW1"], p["mb1"], act="relu", extra_adds=(han,),
                 out_sum=True)
    r2 = _mm(y, p["mW2"], p["mb2"])
    t3, s3 = _add_stats(r2, out)
    return _bn_apply(t3, s3, p["n3_g"], p["n3_b"])


def _bn_relu(x, g, b):
    _, s = _add_stats(x)
    return _bn_apply(x, s, g, b, relu=True)


def kernel(x, edge_attr, params, edge_index):
    p1, p2 = params["gps1"], params["gps2"]
    src = edge_index[0]
    dst = edge_index[1]
    ea1 = _mm(edge_attr, p1["We"], p1["be"], block_rows=4000)
    ea2 = _mm(edge_attr, p2["We"], p2["be"], block_rows=4000)
    h = _gps_layer(x, ea1, src, dst, p1, heads=2)
    h = _bn_relu(h, params["bn1_g"], params["bn1_b"])
    h = _mm(h, params["lin1_W"], params["lin1_b"])
    h = _bn_relu(h, params["bn2_g"], params["bn2_b"])
    h = _gps_layer(h, ea2, src, dst, p2, heads=1)
    h = _bn_relu(h, params["bn2_g"], params["bn2_b"])
    return _mm(h, params["lin2_W"], params["lin2_b"])
